# Initial kernel scaffold; baseline (speedup 1.0000x reference)
import jax
import jax.numpy as jnp
from jax.experimental import pallas as pl


def kernel(x, W1, g1, b1, W2, g2, b2, W3, g3, b3, W4, g4, b4, W5, g5, b5):
    raise NotImplementedError("write your pallas kernel here")


# speedup vs baseline: 5.8537x; 5.8537x over previous
import functools

import jax
import jax.numpy as jnp
from jax import lax
from jax.experimental import pallas as pl

KNN = 20
EPS = 1e-5
NEG_INF = float("-inf")


def _leaky(v):
    return jnp.where(v >= 0, v, 0.2 * v)


def _dot(a, b, dims, precision=lax.Precision.DEFAULT):
    return lax.dot_general(a, b, (dims, ((), ())),
                           precision=precision,
                           preferred_element_type=jnp.float32)


def _split3(v):
    v1 = v.astype(jnp.bfloat16)
    r = v - v1.astype(jnp.float32)
    v2 = r.astype(jnp.bfloat16)
    v3 = (r - v2.astype(jnp.float32)).astype(jnp.bfloat16)
    return v1, v2, v3


def _layer_body(xt_ref, w_ref, raw_ref, s1_ref, s2_ref, *, n):
    b = pl.program_id(0)
    xt = xt_ref[0]
    inner2 = -2.0 * _dot(xt, xt, ((0,), (0,)))
    xx_row = jnp.sum(xt * xt, axis=0, keepdims=True)
    xx_col = xx_row.T
    pd = (-xx_col - inner2) - xx_row
    iota = lax.broadcasted_iota(jnp.int32, (n, n), 1)

    xs1, xs2, xs3 = _split3(xt)
    w = w_ref[...]
    mx = None
    sm = None
    sq = None
    for k in range(KNN):
        m = jnp.max(pd, axis=1, keepdims=True)
        jsel = jnp.min(jnp.where(pd == m, iota, n), axis=1, keepdims=True)
        sel = iota == jsel
        selb = sel.astype(jnp.bfloat16)
        g = (_dot(xs1, selb, ((1,), (1,)))
             + _dot(xs2, selb, ((1,), (1,)))
             + _dot(xs3, selb, ((1,), (1,))))
        h = jnp.concatenate([g - xt, xt], axis=0)
        out = _dot(w, h, ((1,), (0,)))
        mx = out if k == 0 else jnp.maximum(mx, out)
        sm = out if k == 0 else sm + out
        sq = out * out if k == 0 else sq + out * out
        pd = jnp.where(sel, NEG_INF, pd)

    raw_ref[0] = mx
    p1 = jnp.sum(sm, axis=1, keepdims=True)
    p2 = jnp.sum(sq, axis=1, keepdims=True)

    @pl.when(b == 0)
    def _():
        s1_ref[...] = jnp.zeros_like(s1_ref)
        s2_ref[...] = jnp.zeros_like(s2_ref)

    s1_ref[...] += p1
    s2_ref[...] += p2


def _layer(xt, w):
    bsz, c, n = xt.shape
    o = w.shape[0]
    return pl.pallas_call(
        functools.partial(_layer_body, n=n),
        grid=(bsz,),
        in_specs=[
            pl.BlockSpec((1, c, n), lambda b: (b, 0, 0)),
            pl.BlockSpec((o, 2 * c), lambda b: (0, 0)),
        ],
        out_specs=[
            pl.BlockSpec((1, o, n), lambda b: (b, 0, 0)),
            pl.BlockSpec((o, 1), lambda b: (0, 0)),
            pl.BlockSpec((o, 1), lambda b: (0, 0)),
        ],
        out_shape=[
            jax.ShapeDtypeStruct((bsz, o, n), jnp.float32),
            jax.ShapeDtypeStruct((o, 1), jnp.float32),
            jax.ShapeDtypeStruct((o, 1), jnp.float32),
        ],
    )(xt, w)


def _norm_body(raw_ref, s1_ref, s2_ref, g_ref, b_ref, out_ref, *, count):
    mean = s1_ref[...] / count
    var = s2_ref[...] / count - mean * mean
    inv = lax.rsqrt(var + EPS)
    out_ref[0] = _leaky((raw_ref[0] - mean) * inv * g_ref[...] + b_ref[...])


def _norm(raw, s1, s2, g, b, count):
    bsz, o, n = raw.shape
    return pl.pallas_call(
        functools.partial(_norm_body, count=float(count)),
        grid=(bsz,),
        in_specs=[
            pl.BlockSpec((1, o, n), lambda i: (i, 0, 0)),
            pl.BlockSpec((o, 1), lambda i: (0, 0)),
            pl.BlockSpec((o, 1), lambda i: (0, 0)),
            pl.BlockSpec((o, 1), lambda i: (0, 0)),
            pl.BlockSpec((o, 1), lambda i: (0, 0)),
        ],
        out_specs=pl.BlockSpec((1, o, n), lambda i: (i, 0, 0)),
        out_shape=jax.ShapeDtypeStruct((bsz, o, n), jnp.float32),
    )(raw, s1, s2, g.reshape(o, 1), b.reshape(o, 1))


def _final_body(x1_ref, x2_ref, x3_ref, x4_ref, w_ref, raw_ref, s1_ref, s2_ref):
    b = pl.program_id(0)
    cat = jnp.concatenate(
        [x1_ref[0], x2_ref[0], x3_ref[0], x4_ref[0]], axis=0)
    h = _dot(w_ref[...], cat, ((1,), (0,)))
    raw_ref[0] = h
    p1 = jnp.sum(h, axis=1, keepdims=True)
    p2 = jnp.sum(h * h, axis=1, keepdims=True)

    @pl.when(b == 0)
    def _():
        s1_ref[...] = jnp.zeros_like(s1_ref)
        s2_ref[...] = jnp.zeros_like(s2_ref)

    s1_ref[...] += p1
    s2_ref[...] += p2


def _final(x1, x2, x3, x4, w):
    bsz, _, n = x1.shape
    o = w.shape[0]
    specs = [pl.BlockSpec((1, x.shape[1], n), lambda b: (b, 0, 0))
             for x in (x1, x2, x3, x4)]
    return pl.pallas_call(
        _final_body,
        grid=(bsz,),
        in_specs=specs + [pl.BlockSpec(w.shape, lambda b: (0, 0))],
        out_specs=[
            pl.BlockSpec((1, o, n), lambda b: (b, 0, 0)),
            pl.BlockSpec((o, 1), lambda b: (0, 0)),
            pl.BlockSpec((o, 1), lambda b: (0, 0)),
        ],
        out_shape=[
            jax.ShapeDtypeStruct((bsz, o, n), jnp.float32),
            jax.ShapeDtypeStruct((o, 1), jnp.float32),
            jax.ShapeDtypeStruct((o, 1), jnp.float32),
        ],
    )(x1, x2, x3, x4, w)


def _pad_w(w, c, cpad):
    o = w.shape[0]
    wp = jnp.zeros((o, 2 * cpad), w.dtype)
    wp = wp.at[:, :c].set(w[:, :c])
    wp = wp.at[:, cpad:cpad + c].set(w[:, c:])
    return wp


def kernel(x, W1, g1, b1, W2, g2, b2, W3, g3, b3, W4, g4, b4, W5, g5, b5):
    bsz, _, n = x.shape
    cnt = float(bsz * n * KNN)

    x0 = jnp.pad(x, ((0, 0), (0, 5), (0, 0)))

    raw, s1, s2 = _layer(x0, _pad_w(W1, 3, 8))
    x1 = _norm(raw, s1, s2, g1, b1, cnt)

    raw, s1, s2 = _layer(x1, W2)
    x2 = _norm(raw, s1, s2, g2, b2, cnt)

    raw, s1, s2 = _layer(x2, W3)
    x3 = _norm(raw, s1, s2, g3, b3, cnt)

    raw, s1, s2 = _layer(x3, W4)
    x4 = _norm(raw, s1, s2, g4, b4, cnt)

    raw, s1, s2 = _final(x1, x2, x3, x4, W5)
    return _norm(raw, s1, s2, g5, b5, float(bsz * n))
